# baseline (speedup 1.0000x reference)
from functools import partial

import jax
import jax.numpy as jnp
from jax.experimental import pallas as pl
from jax.experimental.pallas import tpu as pltpu

_EPS = 1e-6


def _gem_kernel(p_ref, x_ref, o_ref, *, inv_hw):
    p = p_ref[0]
    nb, hw, C = x_ref.shape
    xc = jnp.maximum(x_ref[...], _EPS)
    t = jnp.exp2(p * jnp.log2(xc)).reshape(nb * hw, C)
    r = jax.lax.broadcasted_iota(jnp.int32, (nb, nb * hw), 1)
    b = jax.lax.broadcasted_iota(jnp.int32, (nb, nb * hw), 0)
    sel = (r // hw == b).astype(jnp.float32)
    s = jax.lax.dot_general(sel, t, (((1,), (0,)), ((), ())),
                            preferred_element_type=jnp.float32) * inv_hw
    o_ref[...] = jnp.exp2(jnp.log2(s) / p)


def _gem(xt, p_arr, nb):
    N, hw, C = xt.shape
    grid = N // nb
    return pl.pallas_call(
        partial(_gem_kernel, inv_hw=1.0 / hw),
        out_shape=jax.ShapeDtypeStruct((N, C), jnp.float32),
        grid=(grid,),
        in_specs=[
            pl.BlockSpec(memory_space=pltpu.MemorySpace.SMEM),
            pl.BlockSpec((nb, hw, C), lambda i: (i, 0, 0)),
        ],
        out_specs=pl.BlockSpec((nb, C), lambda i: (i, 0)),
        compiler_params=pltpu.CompilerParams(
            dimension_semantics=("arbitrary",),
        ),
        cost_estimate=pl.CostEstimate(
            flops=5 * N * C * hw,
            transcendentals=2 * N * C * hw,
            bytes_accessed=N * C * hw * 4 + N * C * 4,
        ),
    )(p_arr, xt)


def kernel(x, p):
    N, C, H, W = x.shape
    hw = H * W
    xt = jnp.transpose(x, (0, 2, 3, 1)).reshape(N, hw, C).astype(jnp.float32)
    p_arr = jnp.asarray(p, jnp.float32).reshape(1)

    nb = 8
    while N % nb:
        nb //= 2

    out = _gem(xt, p_arr, nb)
    return out.reshape(N, C, 1, 1).astype(x.dtype)


# speedup vs baseline: 1.1318x; 1.1318x over previous
from functools import partial

import jax
import jax.numpy as jnp
from jax.experimental import pallas as pl
from jax.experimental.pallas import tpu as pltpu

_EPS = 1e-6


def _gem_kernel(p_ref, x_ref, o_ref, *, inv_hw):
    p = p_ref[0]
    nb, hw, C = x_ref.shape
    xc = jnp.maximum(x_ref[...], _EPS)
    t = jnp.exp2(p * jnp.log2(xc)).reshape(nb * hw, C)
    r = jax.lax.broadcasted_iota(jnp.int32, (nb, nb * hw), 1)
    b = jax.lax.broadcasted_iota(jnp.int32, (nb, nb * hw), 0)
    sel = (r // hw == b).astype(jnp.float32)
    s = jax.lax.dot_general(sel, t, (((1,), (0,)), ((), ())),
                            preferred_element_type=jnp.float32) * inv_hw
    o_ref[...] = jnp.exp2(jnp.log2(s) / p)


def _gem(xt, p_arr, nb):
    N, hw, C = xt.shape
    grid = N // nb
    return pl.pallas_call(
        partial(_gem_kernel, inv_hw=1.0 / hw),
        out_shape=jax.ShapeDtypeStruct((N, C), jnp.float32),
        grid=(grid,),
        in_specs=[
            pl.BlockSpec(memory_space=pltpu.MemorySpace.SMEM),
            pl.BlockSpec((nb, hw, C), lambda i: (i, 0, 0)),
        ],
        out_specs=pl.BlockSpec((nb, C), lambda i: (i, 0)),
        compiler_params=pltpu.CompilerParams(
            dimension_semantics=("arbitrary",),
        ),
        cost_estimate=pl.CostEstimate(
            flops=5 * N * C * hw,
            transcendentals=2 * N * C * hw,
            bytes_accessed=N * C * hw * 4 + N * C * 4,
        ),
    )(p_arr, xt)


def kernel(x, p):
    N, C, H, W = x.shape
    hw = H * W
    xt = jnp.transpose(x, (0, 2, 3, 1)).reshape(N, hw, C).astype(jnp.float32)
    p_arr = jnp.asarray(p, jnp.float32).reshape(1)

    nb = 32
    while N % nb:
        nb //= 2

    out = _gem(xt, p_arr, nb)
    return out.reshape(N, C, 1, 1).astype(x.dtype)


# speedup vs baseline: 1.1346x; 1.0025x over previous
from functools import partial

import jax
import jax.numpy as jnp
from jax.experimental import pallas as pl
from jax.experimental.pallas import tpu as pltpu

_EPS = 1e-6


def _gem_kernel(p_ref, x_ref, o_ref, *, inv_hw):
    p = p_ref[0]
    nb, hw, C = x_ref.shape
    xc = jnp.maximum(x_ref[...], _EPS)
    t = jnp.exp2(p * jnp.log2(xc)).reshape(nb * hw, C)
    r = jax.lax.broadcasted_iota(jnp.int32, (nb, nb * hw), 1)
    b = jax.lax.broadcasted_iota(jnp.int32, (nb, nb * hw), 0)
    sel = (r // hw == b).astype(jnp.float32)
    s = jax.lax.dot_general(sel, t, (((1,), (0,)), ((), ())),
                            preferred_element_type=jnp.float32) * inv_hw
    o_ref[...] = jnp.exp2(jnp.log2(s) / p)


def _gem_kernel2(p_ref, xa_ref, xb_ref, o_ref, *, inv_hw):
    p = p_ref[0]
    nb, hw, cs = xa_ref.shape
    r = jax.lax.broadcasted_iota(jnp.int32, (nb, nb * hw), 1)
    b = jax.lax.broadcasted_iota(jnp.int32, (nb, nb * hw), 0)
    sel = (r // hw == b).astype(jnp.float32)
    for i, x_ref in enumerate((xa_ref, xb_ref)):
        xc = jnp.maximum(x_ref[...], _EPS)
        t = jnp.exp2(p * jnp.log2(xc)).reshape(nb * hw, cs)
        s = jax.lax.dot_general(sel, t, (((1,), (0,)), ((), ())),
                                preferred_element_type=jnp.float32) * inv_hw
        o_ref[:, i * cs:(i + 1) * cs] = jnp.exp2(jnp.log2(s) / p)


def _gem(xt, p_arr, nb):
    N, hw, C = xt.shape
    grid = N // nb
    cs = C // 2
    return pl.pallas_call(
        partial(_gem_kernel2, inv_hw=1.0 / hw),
        out_shape=jax.ShapeDtypeStruct((N, C), jnp.float32),
        grid=(grid,),
        in_specs=[
            pl.BlockSpec(memory_space=pltpu.MemorySpace.SMEM),
            pl.BlockSpec((nb, hw, cs), lambda i: (i, 0, 0)),
            pl.BlockSpec((nb, hw, cs), lambda i: (i, 0, 1)),
        ],
        out_specs=pl.BlockSpec((nb, C), lambda i: (i, 0)),
        compiler_params=pltpu.CompilerParams(
            dimension_semantics=("arbitrary",),
        ),
        cost_estimate=pl.CostEstimate(
            flops=5 * N * C * hw,
            transcendentals=2 * N * C * hw,
            bytes_accessed=N * C * hw * 4 + N * C * 4,
        ),
    )(p_arr, xt, xt)


def kernel(x, p):
    N, C, H, W = x.shape
    hw = H * W
    xt = jnp.transpose(x, (0, 2, 3, 1)).reshape(N, hw, C).astype(jnp.float32)
    p_arr = jnp.asarray(p, jnp.float32).reshape(1)

    nb = 32
    while N % nb:
        nb //= 2

    out = _gem(xt, p_arr, nb)
    return out.reshape(N, C, 1, 1).astype(x.dtype)
